# Initial kernel scaffold; baseline (speedup 1.0000x reference)
import jax
import jax.numpy as jnp
from jax.experimental import pallas as pl


def kernel(user_features, item_features, edge_index, W2, b2, W3, b3, W4, b4, W5, b5, Wp, bp):
    raise NotImplementedError("write your pallas kernel here")


# speedup vs baseline: 3.7937x; 3.7937x over previous
import functools

import jax
import jax.numpy as jnp
from jax import lax
from jax.experimental import pallas as pl
from jax.experimental.pallas import tpu as pltpu
from jax.experimental.pallas import tpu_sc as plsc

N_U = 5000
N_I = 5000
D = 128
NC = 2
NS = 16
NW = NC * NS
NPAD = 5120
RPT = NPAD // NS
CH = 128
CNTW = 16

_INTERP = False
_SC_CNT = True
_SC_MP = True
_SC_SCORE = True
_mesh = plsc.VectorSubcoreMesh(core_axis_name="c", subcore_axis_name="s",
                               num_cores=NC, num_subcores=NS)

f32 = jnp.float32
i32 = jnp.int32


def _fill_zero_2d(ref, rows, cols):
    z = jnp.zeros((16,), f32)

    def body(r, _):
        for j in range(cols // 16):
            ref[r, pl.ds(j * 16, 16)] = z
        return 0

    lax.fori_loop(0, rows, body, 0)


def _fill_ones_2d(ref, rows, cols):
    o = jnp.ones((16,), f32)

    def body(r, _):
        for j in range(cols // 16):
            ref[r, pl.ds(j * 16, 16)] = o
        return 0

    lax.fori_loop(0, rows, body, 0)


def _zero_acc_rows(zbuf, acc, base, nrows, width):
    full, rem = nrows // CH, nrows % CH
    for k in range(full):
        pltpu.sync_copy(zbuf, acc.at[pl.ds(base + k * CH, CH)])
    if rem:
        pltpu.sync_copy(zbuf.at[pl.ds(0, rem)], acc.at[pl.ds(base + full * CH, rem)])


def _make_mp(e_pad):
    ept = e_pad // NW
    nch = ept // CH

    out_type = [
        jax.ShapeDtypeStruct((NC, NPAD, D), f32),
        jax.ShapeDtypeStruct((NC, NPAD, D), f32),
    ]
    scratch = [
        pltpu.VMEM_SHARED((NPAD, D), f32),
        pltpu.VMEM_SHARED((NPAD, D), f32),
        pltpu.VMEM((CH,), i32),
        pltpu.VMEM((CH,), i32),
        pltpu.VMEM((CH,), i32),
        pltpu.VMEM((CH,), i32),
        pltpu.VMEM((CH, D), f32),
        pltpu.VMEM((CH, D), f32),
        pltpu.SemaphoreType.DMA,
        pltpu.SemaphoreType.DMA,
    ]

    def body(utab, itab, src_g, dst_g, src_s, dst_s,
             pA, pB, accA, accB, isg, idg, iss, ids, rA, rB, semA, semB):
        c = lax.axis_index("c")
        s = lax.axis_index("s")

        _fill_zero_2d(rA, CH, D)
        base = s * RPT
        _zero_acc_rows(rA, accA, base, RPT, D)
        _zero_acc_rows(rA, accB, base, RPT, D)
        plsc.subcore_barrier()

        ebase = c * (e_pad // NC) + s * ept

        def step(i, _):
            e0 = ebase + i * CH
            pltpu.sync_copy(src_g.at[pl.ds(e0, CH)], isg)
            pltpu.sync_copy(dst_g.at[pl.ds(e0, CH)], idg)
            pltpu.sync_copy(src_s.at[pl.ds(e0, CH)], iss)
            pltpu.sync_copy(dst_s.at[pl.ds(e0, CH)], ids)
            cpA = pltpu.async_copy(utab.at[isg], rA, semA)
            cpB = pltpu.async_copy(itab.at[idg], rB, semB)
            cpA.wait()
            pltpu.sync_copy(rA, accA.at[ids], add=True)
            cpB.wait()
            pltpu.sync_copy(rB, accB.at[iss], add=True)
            return 0

        lax.fori_loop(0, nch, step, 0)
        plsc.subcore_barrier()

        pltpu.sync_copy(accA.at[pl.ds(base, RPT)], pA.at[c, pl.ds(base, RPT)])
        pltpu.sync_copy(accB.at[pl.ds(base, RPT)], pB.at[c, pl.ds(base, RPT)])

    return pl.kernel(body, out_type=out_type, mesh=_mesh, scratch_types=scratch, interpret=_INTERP)


def _make_counts(e_pad):
    ept = e_pad // NW
    nch = ept // 16

    out_type = [
        jax.ShapeDtypeStruct((NC, NS, NPAD), f32),
        jax.ShapeDtypeStruct((NC, NS, NPAD), f32),
    ]
    scratch = [
        pltpu.VMEM((NPAD,), f32),
        pltpu.VMEM((NPAD,), f32),
        pltpu.VMEM((ept,), i32),
        pltpu.VMEM((ept,), i32),
    ]

    def body(src_s, dst_s, cA, cB, hd, hs, sv, dv):
        c = lax.axis_index("c")
        s = lax.axis_index("s")
        ebase = c * (e_pad // NC) + s * ept
        pltpu.sync_copy(src_s.at[pl.ds(ebase, ept)], sv)
        pltpu.sync_copy(dst_s.at[pl.ds(ebase, ept)], dv)

        def z(j, _):
            hd[pl.ds(j * 16, 16)] = jnp.zeros((16,), f32)
            hs[pl.ds(j * 16, 16)] = jnp.zeros((16,), f32)
            return 0

        lax.fori_loop(0, NPAD // 16, z, 0)
        ones = jnp.ones((16,), f32)

        def step(i, _):
            plsc.addupdate_scatter(hd, [dv[pl.ds(i * 16, 16)]], ones)
            plsc.addupdate_scatter(hs, [sv[pl.ds(i * 16, 16)]], ones)
            return 0

        lax.fori_loop(0, nch, step, 0)
        pltpu.sync_copy(hd, cA.at[c, s])
        pltpu.sync_copy(hs, cB.at[c, s])

    return pl.kernel(
        body, out_type=out_type, mesh=_mesh, scratch_types=scratch,
        compiler_params=pltpu.CompilerParams(needs_layout_passes=False),
        interpret=_INTERP)


def _tc0_body(uf, itf, pu, pi, cu, ci, w2a, w2b, b2, w3a, w3b, b3, uf1, itf1):
    dn = (((1,), (1,)), ((), ()))
    degu = jnp.clip(jnp.sum(cu[...], axis=(0, 1))[:N_U], 1.0, None)
    degi = jnp.clip(jnp.sum(ci[...], axis=(0, 1))[:N_I], 1.0, None)
    hu = (pu[0, :N_U, :] + pu[1, :N_U, :]) / degu[:, None]
    hi = (pi[0, :N_I, :] + pi[1, :N_I, :]) / degi[:, None]
    uf1[...] = (lax.dot_general(uf[...], w2a[...], dn, preferred_element_type=f32)
                + lax.dot_general(hu, w2b[...], dn, preferred_element_type=f32)
                + b2[...][None, :])
    itf1[...] = (lax.dot_general(itf[...], w3a[...], dn, preferred_element_type=f32)
                 + lax.dot_general(hi, w3b[...], dn, preferred_element_type=f32)
                 + b3[...][None, :])


def _tc1_body(uf1, itf1, pu, pi, cu, ci, w4a, w4b, b4, w5a, w5b, b5,
              wpu, wpi, bp, su, si):
    dn_nt = (((1,), (1,)), ((), ()))
    dn_nn = (((1,), (0,)), ((), ()))
    degu = jnp.clip(jnp.sum(cu[...], axis=(0, 1))[:N_U], 1.0, None)
    degi = jnp.clip(jnp.sum(ci[...], axis=(0, 1))[:N_I], 1.0, None)
    hu = (pu[0, :N_U, :] + pu[1, :N_U, :]) / degu[:, None]
    hi = (pi[0, :N_I, :] + pi[1, :N_I, :]) / degi[:, None]
    ku1 = lax.dot_general(wpu[...], w4a[...], dn_nn, preferred_element_type=f32)
    ku2 = lax.dot_general(wpu[...], w4b[...], dn_nn, preferred_element_type=f32)
    ki1 = lax.dot_general(wpi[...], w5a[...], dn_nn, preferred_element_type=f32)
    ki2 = lax.dot_general(wpi[...], w5b[...], dn_nn, preferred_element_type=f32)
    cu_const = (lax.dot_general(wpu[...], b4[...][None, :], dn_nt,
                                preferred_element_type=f32) + bp[...][:, None])
    ci_const = lax.dot_general(wpi[...], b5[...][None, :], dn_nt,
                               preferred_element_type=f32)
    su[...] = (lax.dot_general(ku1, uf1[...], dn_nt, preferred_element_type=f32)
               + lax.dot_general(ku2, hu, dn_nt, preferred_element_type=f32)
               + cu_const)
    si[...] = (lax.dot_general(ki1, itf1[...], dn_nt, preferred_element_type=f32)
               + lax.dot_general(ki2, hi, dn_nt, preferred_element_type=f32)
               + ci_const)


def _make_score(e_pad):
    ept = e_pad // NW
    nch = ept // 16

    out_type = jax.ShapeDtypeStruct((e_pad * 2,), f32)
    scratch = [
        pltpu.VMEM((N_U,), f32),
        pltpu.VMEM((N_U,), f32),
        pltpu.VMEM((N_I,), f32),
        pltpu.VMEM((N_I,), f32),
        pltpu.VMEM((ept,), i32),
        pltpu.VMEM((ept,), i32),
        pltpu.VMEM((ept * 2,), f32),
    ]

    def body(su, si, src_g, dst_g, out, su0, su1, si0, si1, sv, dv, ov):
        c = lax.axis_index("c")
        s = lax.axis_index("s")
        ebase = c * (e_pad // NC) + s * ept
        pltpu.sync_copy(su.at[0], su0)
        pltpu.sync_copy(su.at[1], su1)
        pltpu.sync_copy(si.at[0], si0)
        pltpu.sync_copy(si.at[1], si1)
        pltpu.sync_copy(src_g.at[pl.ds(ebase, ept)], sv)
        pltpu.sync_copy(dst_g.at[pl.ds(ebase, ept)], dv)
        iota = lax.iota(i32, 16)

        def step(i, _):
            sidx = sv[pl.ds(i * 16, 16)]
            didx = dv[pl.ds(i * 16, 16)]
            s0 = plsc.load_gather(su0, [sidx]) + plsc.load_gather(si0, [didx])
            s1 = plsc.load_gather(su1, [sidx]) + plsc.load_gather(si1, [didx])
            pos = i * 32 + iota * 2
            plsc.store_scatter(ov, [pos], s0)
            plsc.store_scatter(ov, [pos + 1], s1)
            return 0

        lax.fori_loop(0, nch, step, 0)
        pltpu.sync_copy(ov, out.at[pl.ds(ebase * 2, ept * 2)])

    return pl.kernel(
        body, out_type=out_type, mesh=_mesh, scratch_types=scratch,
        compiler_params=pltpu.CompilerParams(needs_layout_passes=False),
        interpret=_INTERP)


def kernel(user_features, item_features, edge_index,
           W2, b2, W3, b3, W4, b4, W5, b5, Wp, bp):
    E = edge_index.shape[1]
    grain = NW * CH
    e_pad = ((E + grain - 1) // grain) * grain
    pad = e_pad - E

    src = edge_index[0]
    dst = edge_index[1]
    zpad = jnp.zeros((pad,), i32)
    jpad = jnp.full((pad,), N_U, i32)
    src_g = jnp.concatenate([src, zpad])
    dst_g = jnp.concatenate([dst, zpad])
    src_s = jnp.concatenate([src, jpad])
    dst_s = jnp.concatenate([dst, jpad])

    w2a, w2b = W2[:, :D], W2[:, D:]
    w3a, w3b = W3[:, :D], W3[:, D:]
    w4a, w4b = W4[:, :D], W4[:, D:]
    w5a, w5b = W5[:, :D], W5[:, D:]
    wpu, wpi = Wp[:, :D], Wp[:, D:]

    mp = _make_mp(e_pad)
    counts_k = _make_counts(e_pad)
    score_k = _make_score(e_pad)

    def _jnp_counts():
        ci_ = jnp.zeros((NC, NS, NPAD), f32)
        ci_ = ci_.at[0, 0, :].add(
            jax.ops.segment_sum(jnp.ones((e_pad,), f32), dst_s, num_segments=NPAD))
        cu_ = jnp.zeros((NC, NS, NPAD), f32)
        cu_ = cu_.at[0, 0, :].add(
            jax.ops.segment_sum(jnp.ones((e_pad,), f32), src_s, num_segments=NPAD))
        return ci_, cu_

    def _jnp_mp(utab, itab):
        pi_ = jnp.zeros((NC, NPAD, D), f32)
        pi_ = pi_.at[0].add(jax.ops.segment_sum(utab[src_g], dst_s, num_segments=NPAD))
        pu_ = jnp.zeros((NC, NPAD, D), f32)
        pu_ = pu_.at[0].add(jax.ops.segment_sum(itab[dst_g], src_s, num_segments=NPAD))
        return pi_, pu_

    if _SC_CNT:
        ci, cu = counts_k(src_s, dst_s)
    else:
        ci, cu = _jnp_counts()
    if _SC_MP:
        pi, pu = mp(user_features, item_features, src_g, dst_g, src_s, dst_s)
    else:
        pi, pu = _jnp_mp(user_features, item_features)

    tc0 = pl.pallas_call(
        _tc0_body,
        interpret=_INTERP,
        out_shape=[jax.ShapeDtypeStruct((N_U, D), f32),
                   jax.ShapeDtypeStruct((N_I, D), f32)],
    )
    uf1, itf1 = tc0(user_features, item_features, pu, pi, cu, ci,
                    w2a, w2b, b2, w3a, w3b, b3)

    if _SC_MP:
        pi2, pu2 = mp(uf1, itf1, src_g, dst_g, src_s, dst_s)
    else:
        pi2, pu2 = _jnp_mp(uf1, itf1)

    tc1 = pl.pallas_call(
        _tc1_body,
        interpret=_INTERP,
        out_shape=[jax.ShapeDtypeStruct((2, N_U), f32),
                   jax.ShapeDtypeStruct((2, N_I), f32)],
    )
    su, si = tc1(uf1, itf1, pu2, pi2, cu, ci,
                 w4a, w4b, b4, w5a, w5b, b5, wpu, wpi, bp)

    if _SC_SCORE:
        flat = score_k(su, si, src_g, dst_g)
        return flat.reshape(e_pad, 2)[:E]
    return (su[:, src_g] + si[:, dst_g]).T[:E]


# speedup vs baseline: 4.8967x; 1.2908x over previous
import functools

import jax
import jax.numpy as jnp
from jax import lax
from jax.experimental import pallas as pl
from jax.experimental.pallas import tpu as pltpu
from jax.experimental.pallas import tpu_sc as plsc

N_U = 5000
N_I = 5000
D = 128
NC = 2
NS = 16
NW = NC * NS
NPAD = 5120
RPT = NPAD // NS
CH = 128
CNTW = 16

_INTERP = False
_SC_CNT = True
_SC_MP = True
_SC_SCORE = True
_mesh = plsc.VectorSubcoreMesh(core_axis_name="c", subcore_axis_name="s",
                               num_cores=NC, num_subcores=NS)

f32 = jnp.float32
i32 = jnp.int32


def _fill_zero_2d(ref, rows, cols):
    z = jnp.zeros((16,), f32)

    def body(r, _):
        for j in range(cols // 16):
            ref[r, pl.ds(j * 16, 16)] = z
        return 0

    lax.fori_loop(0, rows, body, 0)


def _fill_ones_2d(ref, rows, cols):
    o = jnp.ones((16,), f32)

    def body(r, _):
        for j in range(cols // 16):
            ref[r, pl.ds(j * 16, 16)] = o
        return 0

    lax.fori_loop(0, rows, body, 0)


def _zero_acc_rows(zbuf, acc, base, nrows, width):
    full, rem = nrows // CH, nrows % CH
    for k in range(full):
        pltpu.sync_copy(zbuf, acc.at[pl.ds(base + k * CH, CH)])
    if rem:
        pltpu.sync_copy(zbuf.at[pl.ds(0, rem)], acc.at[pl.ds(base + full * CH, rem)])


def _make_mp(e_pad):
    ept = e_pad // NS
    nch = ept // CH

    out_type = jax.ShapeDtypeStruct((2, NPAD, D), f32)
    scratch = [
        pltpu.VMEM_SHARED((NPAD, D), f32),
        pltpu.VMEM((2, 2, CH), i32),
        pltpu.VMEM((CH, D), f32),
        pltpu.VMEM((CH, D), f32),
        pltpu.SemaphoreType.DMA,
        pltpu.SemaphoreType.DMA,
        pltpu.SemaphoreType.DMA,
        pltpu.SemaphoreType.DMA,
    ]

    def body(tab2, idx2, out, acc, ibuf, r0, r1, sg0, sg1, ss0, ss1):
        c = lax.axis_index("c")
        s = lax.axis_index("s")

        _fill_zero_2d(r0, CH, D)
        base = s * RPT
        _zero_acc_rows(r0, acc, base, RPT, D)
        plsc.subcore_barrier()

        ebase = s * ept
        tab = tab2.at[c]

        def load_idx(b, i):
            pltpu.sync_copy(idx2.at[c, :, pl.ds(ebase + i * CH, CH)], ibuf.at[b])

        def gather(b, rbuf, sem):
            pltpu.async_copy(tab.at[ibuf.at[b, 0]], rbuf, sem)

        def gather_wait(b, rbuf, sem):
            pltpu.make_async_copy(tab.at[ibuf.at[b, 0]], rbuf, sem).wait()

        def scatter(b, rbuf, sem):
            pltpu.async_copy(rbuf, acc.at[ibuf.at[b, 1]], sem, add=True)

        def scatter_wait(b, rbuf, sem):
            pltpu.make_async_copy(rbuf, acc.at[ibuf.at[b, 1]], sem).wait()

        load_idx(0, 0)
        gather(0, r0, sg0)
        load_idx(1, 1)
        gather(1, r1, sg1)

        def pairstep(k, _):
            i = 2 * k
            gather_wait(0, r0, sg0)
            scatter(0, r0, ss0)
            gather_wait(1, r1, sg1)
            scatter(1, r1, ss1)
            scatter_wait(0, r0, ss0)
            load_idx(0, i + 2)
            gather(0, r0, sg0)
            scatter_wait(1, r1, ss1)
            load_idx(1, i + 3)
            gather(1, r1, sg1)
            return 0

        lax.fori_loop(0, nch // 2 - 1, pairstep, 0)
        gather_wait(0, r0, sg0)
        scatter(0, r0, ss0)
        gather_wait(1, r1, sg1)
        scatter(1, r1, ss1)
        scatter_wait(0, r0, ss0)
        scatter_wait(1, r1, ss1)
        plsc.subcore_barrier()

        pltpu.sync_copy(acc.at[pl.ds(base, RPT)], out.at[c, pl.ds(base, RPT)])

    return pl.kernel(body, out_type=out_type, mesh=_mesh, scratch_types=scratch, interpret=_INTERP)


def _make_counts(e_pad):
    ept = e_pad // NW
    nch = ept // 16

    out_type = [
        jax.ShapeDtypeStruct((NC, NS, NPAD), f32),
        jax.ShapeDtypeStruct((NC, NS, NPAD), f32),
    ]
    scratch = [
        pltpu.VMEM((NPAD,), f32),
        pltpu.VMEM((NPAD,), f32),
        pltpu.VMEM((ept,), i32),
        pltpu.VMEM((ept,), i32),
    ]

    def body(src_s, dst_s, cA, cB, hd, hs, sv, dv):
        c = lax.axis_index("c")
        s = lax.axis_index("s")
        ebase = c * (e_pad // NC) + s * ept
        pltpu.sync_copy(src_s.at[pl.ds(ebase, ept)], sv)
        pltpu.sync_copy(dst_s.at[pl.ds(ebase, ept)], dv)

        def z(j, _):
            hd[pl.ds(j * 16, 16)] = jnp.zeros((16,), f32)
            hs[pl.ds(j * 16, 16)] = jnp.zeros((16,), f32)
            return 0

        lax.fori_loop(0, NPAD // 16, z, 0)
        ones = jnp.ones((16,), f32)

        def step(i, _):
            plsc.addupdate_scatter(hd, [dv[pl.ds(i * 16, 16)]], ones)
            plsc.addupdate_scatter(hs, [sv[pl.ds(i * 16, 16)]], ones)
            return 0

        lax.fori_loop(0, nch, step, 0)
        pltpu.sync_copy(hd, cA.at[c, s])
        pltpu.sync_copy(hs, cB.at[c, s])

    return pl.kernel(
        body, out_type=out_type, mesh=_mesh, scratch_types=scratch,
        compiler_params=pltpu.CompilerParams(needs_layout_passes=False),
        interpret=_INTERP)


def _tc0_body(uf, itf, pmp, cu, ci, w2a, w2b, b2, w3a, w3b, b3, uf1, itf1):
    dn = (((1,), (1,)), ((), ()))
    degu = jnp.clip(jnp.sum(cu[...], axis=(0, 1))[:N_U], 1.0, None)
    degi = jnp.clip(jnp.sum(ci[...], axis=(0, 1))[:N_I], 1.0, None)
    hu = pmp[1, :N_U, :] / degu[:, None]
    hi = pmp[0, :N_I, :] / degi[:, None]
    uf1[...] = (lax.dot_general(uf[...], w2a[...], dn, preferred_element_type=f32)
                + lax.dot_general(hu, w2b[...], dn, preferred_element_type=f32)
                + b2[...][None, :])
    itf1[...] = (lax.dot_general(itf[...], w3a[...], dn, preferred_element_type=f32)
                 + lax.dot_general(hi, w3b[...], dn, preferred_element_type=f32)
                 + b3[...][None, :])


def _tc1_body(uf1, itf1, pmp, cu, ci, w4a, w4b, b4, w5a, w5b, b5,
              wpu, wpi, bp, su, si):
    dn_nt = (((1,), (1,)), ((), ()))
    dn_nn = (((1,), (0,)), ((), ()))
    degu = jnp.clip(jnp.sum(cu[...], axis=(0, 1))[:N_U], 1.0, None)
    degi = jnp.clip(jnp.sum(ci[...], axis=(0, 1))[:N_I], 1.0, None)
    hu = pmp[1, :N_U, :] / degu[:, None]
    hi = pmp[0, :N_I, :] / degi[:, None]
    ku1 = lax.dot_general(wpu[...], w4a[...], dn_nn, preferred_element_type=f32)
    ku2 = lax.dot_general(wpu[...], w4b[...], dn_nn, preferred_element_type=f32)
    ki1 = lax.dot_general(wpi[...], w5a[...], dn_nn, preferred_element_type=f32)
    ki2 = lax.dot_general(wpi[...], w5b[...], dn_nn, preferred_element_type=f32)
    cu_const = (lax.dot_general(wpu[...], b4[...][None, :], dn_nt,
                                preferred_element_type=f32) + bp[...][:, None])
    ci_const = lax.dot_general(wpi[...], b5[...][None, :], dn_nt,
                               preferred_element_type=f32)
    su[...] = (lax.dot_general(ku1, uf1[...], dn_nt, preferred_element_type=f32)
               + lax.dot_general(ku2, hu, dn_nt, preferred_element_type=f32)
               + cu_const)
    si[...] = (lax.dot_general(ki1, itf1[...], dn_nt, preferred_element_type=f32)
               + lax.dot_general(ki2, hi, dn_nt, preferred_element_type=f32)
               + ci_const)


def _make_score(e_pad):
    ept = e_pad // NW
    nch = ept // 16

    out_type = jax.ShapeDtypeStruct((e_pad * 2,), f32)
    scratch = [
        pltpu.VMEM((N_U,), f32),
        pltpu.VMEM((N_U,), f32),
        pltpu.VMEM((N_I,), f32),
        pltpu.VMEM((N_I,), f32),
        pltpu.VMEM((ept,), i32),
        pltpu.VMEM((ept,), i32),
        pltpu.VMEM((ept * 2,), f32),
    ]

    def body(su, si, src_g, dst_g, out, su0, su1, si0, si1, sv, dv, ov):
        c = lax.axis_index("c")
        s = lax.axis_index("s")
        ebase = c * (e_pad // NC) + s * ept
        pltpu.sync_copy(su.at[0], su0)
        pltpu.sync_copy(su.at[1], su1)
        pltpu.sync_copy(si.at[0], si0)
        pltpu.sync_copy(si.at[1], si1)
        pltpu.sync_copy(src_g.at[pl.ds(ebase, ept)], sv)
        pltpu.sync_copy(dst_g.at[pl.ds(ebase, ept)], dv)
        iota = lax.iota(i32, 16)

        def step(i, _):
            sidx = sv[pl.ds(i * 16, 16)]
            didx = dv[pl.ds(i * 16, 16)]
            s0 = plsc.load_gather(su0, [sidx]) + plsc.load_gather(si0, [didx])
            s1 = plsc.load_gather(su1, [sidx]) + plsc.load_gather(si1, [didx])
            pos = i * 32 + iota * 2
            plsc.store_scatter(ov, [pos], s0)
            plsc.store_scatter(ov, [pos + 1], s1)
            return 0

        lax.fori_loop(0, nch, step, 0)
        pltpu.sync_copy(ov, out.at[pl.ds(ebase * 2, ept * 2)])

    return pl.kernel(
        body, out_type=out_type, mesh=_mesh, scratch_types=scratch,
        compiler_params=pltpu.CompilerParams(needs_layout_passes=False),
        interpret=_INTERP)


def kernel(user_features, item_features, edge_index,
           W2, b2, W3, b3, W4, b4, W5, b5, Wp, bp):
    E = edge_index.shape[1]
    grain = NS * CH * 2
    e_pad = ((E + grain - 1) // grain) * grain
    pad = e_pad - E

    src = edge_index[0]
    dst = edge_index[1]
    zpad = jnp.zeros((pad,), i32)
    jpad = jnp.full((pad,), N_U, i32)
    src_g = jnp.concatenate([src, zpad])
    dst_g = jnp.concatenate([dst, zpad])
    src_s = jnp.concatenate([src, jpad])
    dst_s = jnp.concatenate([dst, jpad])
    idx2 = jnp.stack([jnp.stack([src_g, dst_s]), jnp.stack([dst_g, src_s])])

    w2a, w2b = W2[:, :D], W2[:, D:]
    w3a, w3b = W3[:, :D], W3[:, D:]
    w4a, w4b = W4[:, :D], W4[:, D:]
    w5a, w5b = W5[:, :D], W5[:, D:]
    wpu, wpi = Wp[:, :D], Wp[:, D:]

    mp = _make_mp(e_pad)
    counts_k = _make_counts(e_pad)
    score_k = _make_score(e_pad)

    def _jnp_counts():
        ci_ = jnp.zeros((NC, NS, NPAD), f32)
        ci_ = ci_.at[0, 0, :].add(
            jax.ops.segment_sum(jnp.ones((e_pad,), f32), dst_s, num_segments=NPAD))
        cu_ = jnp.zeros((NC, NS, NPAD), f32)
        cu_ = cu_.at[0, 0, :].add(
            jax.ops.segment_sum(jnp.ones((e_pad,), f32), src_s, num_segments=NPAD))
        return ci_, cu_

    def _jnp_mp(utab, itab):
        return jnp.stack([
            jax.ops.segment_sum(utab[src_g], dst_s, num_segments=NPAD),
            jax.ops.segment_sum(itab[dst_g], src_s, num_segments=NPAD)])

    if _SC_CNT:
        ci, cu = counts_k(src_s, dst_s)
    else:
        ci, cu = _jnp_counts()
    if _SC_MP:
        pmp1 = mp(jnp.stack([user_features, item_features]), idx2)
    else:
        pmp1 = _jnp_mp(user_features, item_features)

    tc0 = pl.pallas_call(
        _tc0_body,
        interpret=_INTERP,
        out_shape=[jax.ShapeDtypeStruct((N_U, D), f32),
                   jax.ShapeDtypeStruct((N_I, D), f32)],
    )
    uf1, itf1 = tc0(user_features, item_features, pmp1, cu, ci,
                    w2a, w2b, b2, w3a, w3b, b3)

    if _SC_MP:
        pmp2 = mp(jnp.stack([uf1, itf1]), idx2)
    else:
        pmp2 = _jnp_mp(uf1, itf1)

    tc1 = pl.pallas_call(
        _tc1_body,
        interpret=_INTERP,
        out_shape=[jax.ShapeDtypeStruct((2, N_U), f32),
                   jax.ShapeDtypeStruct((2, N_I), f32)],
    )
    su, si = tc1(uf1, itf1, pmp2, cu, ci,
                 w4a, w4b, b4, w5a, w5b, b5, wpu, wpi, bp)

    if _SC_SCORE:
        flat = score_k(su, si, src_g, dst_g)
        return flat.reshape(e_pad, 2)[:E]
    return (su[:, src_g] + si[:, dst_g]).T[:E]
